# Initial kernel scaffold; baseline (speedup 1.0000x reference)
import jax
import jax.numpy as jnp
from jax.experimental import pallas as pl


def kernel(x, params, edge_index, batch, y):
    raise NotImplementedError("write your pallas kernel here")


# speedup vs baseline: 2.1724x; 2.1724x over previous
import functools

import jax
import jax.numpy as jnp
from jax import lax
from jax.experimental import pallas as pl
from jax.experimental.pallas import tpu as pltpu
from jax.experimental.pallas import tpu_sc as plsc

_N = 10000
_E = 640000
_G = 8
_EPS = 1e-5
_RATIO = 0.7


def _make_segmax(C, CPT, ESPLIT, ECH=2000):
    NCG = C // CPT
    ESH = _E // ESPLIT
    NCHUNK = ESH // ECH
    NV = ECH // 16
    assert NCG * ESPLIT == 32

    mesh = plsc.VectorSubcoreMesh(core_axis_name="c", subcore_axis_name="s")

    @functools.partial(
        pl.kernel,
        out_type=jax.ShapeDtypeStruct((ESPLIT * C * _N,), jnp.float32),
        mesh=mesh,
        compiler_params=pltpu.CompilerParams(needs_layout_passes=False),
        scratch_types=[
            pltpu.VMEM((CPT * _N,), jnp.float32),
            pltpu.VMEM((CPT * _N,), jnp.float32),
            pltpu.VMEM((ECH,), jnp.int32),
            pltpu.VMEM((ECH,), jnp.int32),
        ],
    )
    def segmax(h_hbm, src_hbm, dst_hbm, out_hbm, hv, acc, sidx, didx):
        wid = lax.axis_index("s") * 2 + lax.axis_index("c")
        shard = wid // NCG
        cb = (wid % NCG) * CPT
        ebase = shard * ESH

        for c in range(CPT):
            pltpu.sync_copy(h_hbm.at[pl.ds((cb + c) * _N, _N)],
                            hv.at[pl.ds(c * _N, _N)])

        ninf = jnp.full((16,), -jnp.inf, jnp.float32)

        def init_body(i, carry):
            for c in range(CPT):
                acc[pl.ds(c * _N + i * 16, 16)] = ninf
            return carry

        lax.fori_loop(0, _N // 16, init_body, 0)

        def chunk_body(k, carry):
            off = ebase + k * ECH
            pltpu.sync_copy(src_hbm.at[pl.ds(off, ECH)], sidx)
            pltpu.sync_copy(dst_hbm.at[pl.ds(off, ECH)], didx)

            def vec_body(j, c2):
                s = sidx[pl.ds(j * 16, 16)]
                d = didx[pl.ds(j * 16, 16)]
                for c in range(CPT):
                    sc = s + (c * _N) if c else s
                    dc = d + (c * _N) if c else d
                    val = plsc.load_gather(hv, [sc])

                    def rbody(p):
                        cur = plsc.load_gather(acc, [dc])
                        new = jnp.maximum(cur, val)
                        plsc.store_scatter(acc, [dc], new, mask=p)
                        chk = plsc.load_gather(acc, [dc])
                        return p & (chk < val)

                    lax.while_loop(lambda p: jnp.any(p), rbody,
                                   jnp.full((16,), True))
                return c2

            lax.fori_loop(0, NV, vec_body, 0)
            return carry

        lax.fori_loop(0, NCHUNK, chunk_body, 0)

        for c in range(CPT):
            pltpu.sync_copy(acc.at[pl.ds(c * _N, _N)],
                            out_hbm.at[pl.ds((shard * C + cb + c) * _N, _N)])

    return segmax


def _block_tc(h_t, M, W, b, g, be, res_t):
    C = W.shape[1]

    def body(h_ref, m_ref, w_ref, b_ref, g_ref, be_ref, *rest):
        if res_t is None:
            out_ref = rest[0]
            r_ref = None
        else:
            r_ref, out_ref = rest
        h = h_ref[...]
        mm = jnp.max(m_ref[...], axis=0)
        agg = jnp.where(mm == -jnp.inf, 0.0, mm - h)
        con = jnp.concatenate([h, agg], axis=0)
        hp = lax.dot_general(w_ref[...], con, (((0,), (0,)), ((), ())),
                             preferred_element_type=jnp.float32) + b_ref[...]
        mu = jnp.mean(hp, axis=1, keepdims=True)
        var = jnp.mean((hp - mu) ** 2, axis=1, keepdims=True)
        o = jnp.maximum((hp - mu) / jnp.sqrt(var + _EPS) * g_ref[...]
                        + be_ref[...], 0.0)
        if r_ref is not None:
            o = o + r_ref[...]
        out_ref[...] = o

    args = [h_t, M, W, b.reshape(C, 1), g.reshape(C, 1), be.reshape(C, 1)]
    if res_t is not None:
        args.append(res_t)
    return pl.pallas_call(
        body, out_shape=jax.ShapeDtypeStruct((C, _N), jnp.float32))(*args)


def _fusion_pre(feats_t, W, b):
    BN = 1024
    grid = (-(-_N // BN),)

    def body(f_ref, w_ref, b_ref, o_ref):
        o_ref[...] = lax.dot_general(
            w_ref[...], f_ref[...], (((0,), (0,)), ((), ())),
            preferred_element_type=jnp.float32) + b_ref[...]

    return pl.pallas_call(
        body,
        grid=grid,
        in_specs=[pl.BlockSpec((256, BN), lambda i: (0, i)),
                  pl.BlockSpec((256, 1024), lambda i: (0, 0)),
                  pl.BlockSpec((1024, 1), lambda i: (0, 0))],
        out_specs=pl.BlockSpec((1024, BN), lambda i: (0, i)),
        out_shape=jax.ShapeDtypeStruct((1024, _N), jnp.float32),
    )(feats_t, W, b.reshape(-1, 1))


def _fusion_stats(hp):
    def body(h_ref, o_ref):
        h = h_ref[...]
        mu = jnp.mean(h, axis=1, keepdims=True)
        var = jnp.mean((h - mu) ** 2, axis=1, keepdims=True)
        o_ref[0] = mu
        o_ref[1] = var

    return pl.pallas_call(
        body,
        grid=(4,),
        in_specs=[pl.BlockSpec((256, _N), lambda i: (i, 0))],
        out_specs=pl.BlockSpec((2, 256, 1), lambda i: (0, i, 0)),
        out_shape=jax.ShapeDtypeStruct((2, 1024, 1), jnp.float32))(hp)


def _fusion_post(hp, st, g, be, w):
    BN = 1024
    grid = (-(-_N // BN),)

    def body(h_ref, st_ref, g_ref, be_ref, w_ref, f_ref, s_ref):
        mu = st_ref[0]
        var = st_ref[1]
        f = jnp.maximum((h_ref[...] - mu) / jnp.sqrt(var + _EPS) * g_ref[...]
                        + be_ref[...], 0.0)
        f_ref[...] = f
        wv = w_ref[...]
        wn = jnp.sqrt(jnp.sum(wv * wv))
        s_ref[...] = jnp.tanh(jnp.sum(f * wv, axis=0, keepdims=True) / wn)

    return pl.pallas_call(
        body,
        grid=grid,
        in_specs=[pl.BlockSpec((1024, BN), lambda i: (0, i)),
                  pl.BlockSpec((2, 1024, 1), lambda i: (0, 0, 0)),
                  pl.BlockSpec((1024, 1), lambda i: (0, 0)),
                  pl.BlockSpec((1024, 1), lambda i: (0, 0)),
                  pl.BlockSpec((1024, 1), lambda i: (0, 0))],
        out_specs=[pl.BlockSpec((1024, BN), lambda i: (0, i)),
                   pl.BlockSpec((1, BN), lambda i: (0, i))],
        out_shape=[jax.ShapeDtypeStruct((1024, _N), jnp.float32),
                   jax.ShapeDtypeStruct((1, _N), jnp.float32)],
    )(hp, st, g.reshape(-1, 1), be.reshape(-1, 1), w.reshape(-1, 1))


def _topk_keep(score_row, score_col, batch_row, batch_col):
    B = 400

    def body(sr_ref, sc_ref, br_ref, bc_ref, keep_ref):
        sr = sr_ref[...]
        br = br_ref[...]
        gidx = lax.broadcasted_iota(jnp.int32, (_G, _N), 0)
        cntg = jnp.sum(jnp.where(br == gidx, 1.0, 0.0), axis=1, keepdims=True)
        kg = jnp.ceil(_RATIO * cntg)
        jrow = lax.broadcasted_iota(jnp.int32, (1, _N), 1)

        def step(t, carry):
            i0 = t * B
            si = sc_ref[pl.ds(i0, B), :]
            bi = bc_ref[pl.ds(i0, B), :]
            irow = lax.broadcasted_iota(jnp.int32, (B, 1), 0) + i0
            before = (sr > si) | ((sr == si) & (jrow < irow))
            cnt = jnp.sum(jnp.where((br == bi) & before, 1.0, 0.0),
                          axis=1, keepdims=True)
            kb = jnp.zeros((B, 1), jnp.float32)
            for g in range(_G):
                kb = kb + jnp.where(bi == g, kg[g:g + 1, 0:1], 0.0)
            keep_ref[pl.ds(i0, B), :] = jnp.where(cnt < kb, 1.0, 0.0)
            return carry

        lax.fori_loop(0, _N // B, step, 0)

    return pl.pallas_call(
        body, out_shape=jax.ShapeDtypeStruct((_N, 1), jnp.float32))(
            score_row, score_col, batch_row, batch_col)


def _pool(fus, score_row, keep_row, batch_row):
    def body(f_ref, s_ref, k_ref, b_ref, o_ref):
        pooled = f_ref[...] * s_ref[...]
        keep = k_ref[...] > 0.0
        br = b_ref[...]
        for g in range(_G):
            m = (br == g) & keep
            gmax = jnp.max(jnp.where(m, pooled, -jnp.inf), axis=1,
                           keepdims=True)
            gsum = jnp.sum(jnp.where(m, pooled, 0.0), axis=1, keepdims=True)
            cnt = jnp.sum(jnp.where(m, 1.0, 0.0))
            gmean = gsum / jnp.maximum(cnt, 1.0)
            gmin = jnp.min(jnp.where(m, pooled, jnp.inf), axis=1,
                           keepdims=True)
            o_ref[0, :, pl.ds(g, 1)] = gmax
            o_ref[1, :, pl.ds(g, 1)] = gmean
            o_ref[2, :, pl.ds(g, 1)] = gmin

    return pl.pallas_call(
        body,
        grid=(4,),
        in_specs=[pl.BlockSpec((256, _N), lambda i: (i, 0)),
                  pl.BlockSpec((1, _N), lambda i: (0, 0)),
                  pl.BlockSpec((1, _N), lambda i: (0, 0)),
                  pl.BlockSpec((1, _N), lambda i: (0, 0))],
        out_specs=pl.BlockSpec((3, 256, _G), lambda i: (0, i, 0)),
        out_shape=jax.ShapeDtypeStruct((3, 1024, _G), jnp.float32))(
            fus, score_row, keep_row, batch_row)


def _pred(feat, p1, p2, p3):
    def body(f_ref, w1, b1, g1, be1, w2, b2, g2, be2, w3, b3, o_ref):
        def mlp(h, w_ref, b_ref, g_ref, be_ref):
            hp = lax.dot_general(h, w_ref[...], (((1,), (0,)), ((), ())),
                                 preferred_element_type=jnp.float32) + b_ref[...]
            mu = jnp.mean(hp, axis=0, keepdims=True)
            var = jnp.mean((hp - mu) ** 2, axis=0, keepdims=True)
            return jnp.maximum((hp - mu) / jnp.sqrt(var + _EPS) * g_ref[...]
                               + be_ref[...], 0.0)

        h1 = mlp(f_ref[...], w1, b1, g1, be1)
        h2 = mlp(h1, w2, b2, g2, be2)
        o_ref[...] = lax.dot_general(h2, w3[...], (((1,), (0,)), ((), ())),
                                     preferred_element_type=jnp.float32) + b3[...]

    r = lambda a: a.reshape(1, -1)
    return pl.pallas_call(
        body, out_shape=jax.ShapeDtypeStruct((_G, 2), jnp.float32))(
            feat, p1["W"], r(p1["b"]), r(p1["g"]), r(p1["be"]),
            p2["W"], r(p2["b"]), r(p2["g"]), r(p2["be"]),
            p3["W"], r(p3["b"]))


def kernel(x, params, edge_index, batch, y):
    src = edge_index[0]
    dst = edge_index[1]

    h0 = jnp.transpose(x[:, :16])
    seg_head = _make_segmax(16, 1, 2)
    M0 = seg_head(h0.reshape(-1), src, dst).reshape(2, 16, _N)
    ph = params["head"]
    feats = [_block_tc(h0, M0, ph["W"], ph["b"], ph["g"], ph["be"], None)]

    seg64 = _make_segmax(64, 2, 1)
    for blk in params["blocks"]:
        Mi = seg64(feats[-1].reshape(-1), src, dst).reshape(1, 64, _N)
        feats.append(_block_tc(feats[-1], Mi, blk["W"], blk["b"], blk["g"],
                               blk["be"], feats[-1]))

    feats_t = jnp.concatenate(feats, axis=0)
    fp = params["fusion"]
    hp = _fusion_pre(feats_t, fp["W"], fp["b"])
    st = _fusion_stats(hp)
    fus, score = _fusion_post(hp, st, fp["g"], fp["be"], params["pool_w"])

    br = batch.reshape(1, _N)
    keep_col = _topk_keep(score, score.reshape(_N, 1), br,
                          batch.reshape(_N, 1))
    po = _pool(fus, score, keep_col.reshape(1, _N), br)
    feat = jnp.concatenate([po[0].T, po[1].T, po[2].T], axis=1)
    return _pred(feat, params["pred1"], params["pred2"], params["pred3"])
